# baseline (speedup 1.0000x reference)
import functools

import jax
import jax.numpy as jnp
from jax import lax
from jax.experimental import pallas as pl
from jax.experimental.pallas import tpu as pltpu
from jax.experimental.pallas import tpu_sc as plsc

R, C = 1024, 100000
NCORES, NSUB = 2, 16
NW = NCORES * NSUB
RPW = R // NW
NCH = 10
CH = C // NCH
L = 16
U = 5
ITERS = CH // (L * U)

_NEG_INF = float("-inf")
_BIG = 1 << 30


def _chunk_update(buf, slot, ci, state):
    def body(i, st):
        out = []
        for u in range(U):
            rm, rc = st[2 * u], st[2 * u + 1]
            off = (i * U + u) * L
            v = buf[slot, pl.ds(off, L)]
            colb = ci * CH + off
            p = v > rm
            rm = jnp.where(p, v, rm)
            rc = jnp.where(p, colb, rc)
            out.append(rm)
            out.append(rc)
        return tuple(out)

    return lax.fori_loop(0, ITERS, body, tuple(state))


def _finalize(state):
    lanes = lax.iota(jnp.int32, L)
    rm = state[0]
    cols = state[1] + lanes
    for u in range(1, U):
        rm2 = state[2 * u]
        cols2 = state[2 * u + 1] + lanes
        better = (rm2 > rm) | ((rm2 == rm) & (cols2 < cols))
        rm = jnp.where(better, rm2, rm)
        cols = jnp.where(better, cols2, cols)
    m = jnp.max(rm)
    cand = jnp.where(rm == m, cols, _BIG)
    return jnp.min(cand)


def _argmax_rows_sc(x_flat):
    mesh = plsc.VectorSubcoreMesh(core_axis_name="c", subcore_axis_name="s")

    @functools.partial(
        pl.kernel,
        out_type=jax.ShapeDtypeStruct((R,), jnp.int32),
        mesh=mesh,
        compiler_params=pltpu.CompilerParams(
            needs_layout_passes=False, use_tc_tiling_on_sc=False
        ),
        scratch_types=[
            pltpu.VMEM((2, CH), jnp.float32),
            pltpu.VMEM((RPW,), jnp.int32),
            pltpu.SemaphoreType.DMA,
        ],
    )
    def k(x_hbm, out_hbm, buf, out_v, sem):
        cid = lax.axis_index("c")
        sid = lax.axis_index("s")
        wid = sid * NCORES + cid
        row0 = wid * RPW

        neg = jnp.full((L,), _NEG_INF, dtype=jnp.float32)
        zero = jnp.zeros((L,), dtype=jnp.int32)
        lanes = lax.iota(jnp.int32, L)

        def row_body(r, res):
            res_a, res_b = res
            base = (row0 + r) * C
            st = (neg, zero) * U
            for ci in range(NCH):
                pltpu.sync_copy(x_hbm.at[pl.ds(base + ci * CH, CH)], buf.at[0])
                st = _chunk_update(buf, 0, ci, st)
            idx = _finalize(st)
            res_a = jnp.where(lanes == r, idx, res_a)
            res_b = jnp.where(lanes == r - L, idx, res_b)
            return res_a, res_b

        res_a, res_b = lax.fori_loop(0, RPW, row_body, (zero, zero))
        out_v[pl.ds(0, L)] = res_a
        out_v[pl.ds(L, L)] = res_b

        pltpu.sync_copy(out_v, out_hbm.at[pl.ds(row0, RPW)])

    return k(x_flat)


def kernel(inputs):
    return _argmax_rows_sc(inputs.reshape(-1))


# speedup vs baseline: 2.0739x; 2.0739x over previous
import functools

import jax
import jax.numpy as jnp
from jax import lax
from jax.experimental import pallas as pl
from jax.experimental.pallas import tpu as pltpu
from jax.experimental.pallas import tpu_sc as plsc

R, C = 1024, 100000
NCORES, NSUB = 2, 16
NW = NCORES * NSUB
L = 16
LANE = 128
G = 4
TPC = 11
CW = TPC * LANE
NCHG = 71
CTAIL = C - NCHG * CW
TOTCH = G * NCHG
PAIRS = TOTCH // 2

_NEG_INF = float("-inf")
_BIG = 1 << 30


def _update(v, colb, rm, rc):
    p = v > rm
    return jnp.where(p, v, rm), jnp.where(p, colb, rc)


def _row_result(rm, rc, lanes):
    m = jnp.max(rm)
    cand = jnp.where(rm == m, rc + lanes, _BIG)
    return jnp.min(cand)


def _argmax_rows_sc(x):
    mesh = plsc.VectorSubcoreMesh(core_axis_name="c", subcore_axis_name="s")

    @functools.partial(
        pl.kernel,
        out_type=jax.ShapeDtypeStruct((NW, LANE), jnp.int32),
        mesh=mesh,
        compiler_params=pltpu.CompilerParams(
            needs_layout_passes=False, use_tc_tiling_on_sc=True
        ),
        scratch_types=[
            pltpu.VMEM((2, 8, CW), jnp.float32),
            pltpu.VMEM((G, 8, CTAIL), jnp.float32),
            pltpu.VMEM((1, LANE), jnp.int32),
            pltpu.SemaphoreType.DMA,
            pltpu.SemaphoreType.DMA,
            pltpu.SemaphoreType.DMA,
        ],
    )
    def k(x_hbm, out_hbm, buf, tbuf, outv, sem0, sem1, semt):
        cid = lax.axis_index("c")
        sid = lax.axis_index("s")
        wid = sid * NCORES + cid

        lanes = lax.iota(jnp.int32, L)
        neg = jnp.full((L,), _NEG_INF, dtype=jnp.float32)
        zero = jnp.zeros((L,), dtype=jnp.int32)

        def chunk_src(c):
            g_local = c // NCHG
            jj = c % NCHG
            rowbase = (wid * G + g_local) * 8
            return x_hbm.at[pl.ds(rowbase, 8), pl.ds(jj * CW, CW)]

        def chunk_copy(c, slot):
            sem = sem0 if slot == 0 else sem1
            return pltpu.make_async_copy(chunk_src(c), buf.at[slot], sem)

        def tail_copy(g):
            rowbase = (wid * G + g) * 8
            src = x_hbm.at[pl.ds(rowbase, 8), pl.ds(NCHG * CW, CTAIL)]
            return pltpu.make_async_copy(src, tbuf.at[g], semt)

        for g in range(G):
            tail_copy(g).start()
        chunk_copy(0, 0).start()
        chunk_copy(1, 1).start()
        for g in range(G):
            tail_copy(g).wait()

        def compute_chunk(slot, colbase, st):
            def body(j, st_):
                out = list(st_)
                for s in range(8):
                    rm, rc = out[2 * s], out[2 * s + 1]
                    for q in range(8):
                        off = q * L
                        v = buf[slot, s, pl.ds(j * LANE + off, L)]
                        colb = colbase + j * LANE + off
                        rm, rc = _update(v, colb, rm, rc)
                    out[2 * s], out[2 * s + 1] = rm, rc
                return tuple(out)

            return lax.fori_loop(0, TPC, body, tuple(st))

        def tail_finalize(g, st, acc_a, acc_b):
            for s in range(8):
                rm, rc = st[2 * s], st[2 * s + 1]
                for q in range(CTAIL // L):
                    v = tbuf[g, s, pl.ds(q * L, L)]
                    rm, rc = _update(v, NCHG * CW + q * L, rm, rc)
                idx = _row_result(rm, rc, lanes)
                rl = g * 8 + s
                acc_a = jnp.where(lanes == rl, idx, acc_a)
                acc_b = jnp.where(lanes == rl - L, idx, acc_b)
            return acc_a, acc_b

        def process(c, colbase, slot, carry):
            st, acc_a, acc_b = carry[:-2], carry[-2], carry[-1]
            is_bound = (c % NCHG == 0) & (c > 0)
            acc_a, acc_b = lax.cond(
                is_bound,
                lambda: tail_finalize(c // NCHG - 1, st, acc_a, acc_b),
                lambda: (acc_a, acc_b),
            )
            reset = jnp.full((L,), c % NCHG == 0)
            st = tuple(
                jnp.where(reset, neg if i % 2 == 0 else zero, st[i])
                for i in range(16)
            )
            st = compute_chunk(slot, colbase, st)
            return st + (acc_a, acc_b)

        def pair_body(t, carry):
            c0 = 2 * t
            chunk_copy(c0, 0).wait()
            carry = process(c0, (c0 % NCHG) * CW, 0, carry)

            @pl.when(t < PAIRS - 1)
            def _():
                chunk_copy(c0 + 2, 0).start()

            c1 = c0 + 1
            chunk_copy(c1, 1).wait()
            carry = process(c1, (c1 % NCHG) * CW, 1, carry)

            @pl.when(t < PAIRS - 1)
            def _():
                chunk_copy(c1 + 2, 1).start()

            return carry

        init = (neg, zero) * 8 + (zero, zero)
        carry = lax.fori_loop(0, PAIRS, pair_body, init)
        st, acc_a, acc_b = carry[:-2], carry[-2], carry[-1]
        acc_a, acc_b = tail_finalize(G - 1, st, acc_a, acc_b)

        outv[0, pl.ds(0, L)] = acc_a
        outv[0, pl.ds(L, L)] = acc_b
        pltpu.sync_copy(outv, out_hbm.at[pl.ds(wid, 1)])

    return k(x)


def kernel(inputs):
    out = _argmax_rows_sc(inputs)
    return out[:, : R // NW].reshape(R)


# speedup vs baseline: 2.2463x; 1.0831x over previous
import functools

import jax
import jax.numpy as jnp
from jax import lax
from jax.experimental import pallas as pl
from jax.experimental.pallas import tpu as pltpu

R, C = 1024, 100000
LANE = 128
SUB = 8
NB = R // LANE
RT = 800
NJ = C // RT
_NEG_INF = float("-inf")
_BIG = 1 << 30


def _tc_body(x_ref, o_ref, m_ref, c_ref):
    j = pl.program_id(1)

    @pl.when(j == 0)
    def _():
        m_ref[...] = jnp.full((SUB, LANE), _NEG_INF, dtype=jnp.float32)
        c_ref[...] = jnp.zeros((SUB, LANE), dtype=jnp.int32)

    m = m_ref[...]
    c = c_ref[...]
    for k in range(RT // SUB):
        v = x_ref[pl.ds(SUB * k, SUB), :]
        p = v > m
        m = jnp.where(p, v, m)
        c = jnp.where(p, j * RT + SUB * k, c)
    m_ref[...] = m
    c_ref[...] = c

    @pl.when(j == NJ - 1)
    def _():
        mm = jnp.max(m, axis=0, keepdims=True)
        srow = lax.broadcasted_iota(jnp.int32, (SUB, LANE), 0)
        cand = jnp.where(m == mm, c + srow, _BIG)
        o_ref[...] = jnp.min(cand, axis=0, keepdims=True)[None]


def _argmax_tc(xt):
    return pl.pallas_call(
        _tc_body,
        grid=(NB, NJ),
        in_specs=[pl.BlockSpec((RT, LANE), lambda b, j: (j, b))],
        out_specs=pl.BlockSpec((1, 1, LANE), lambda b, j: (b, 0, 0)),
        out_shape=jax.ShapeDtypeStruct((NB, 1, LANE), jnp.int32),
        scratch_shapes=[
            pltpu.VMEM((SUB, LANE), jnp.float32),
            pltpu.VMEM((SUB, LANE), jnp.int32),
        ],
        compiler_params=pltpu.CompilerParams(
            dimension_semantics=("parallel", "arbitrary"),
        ),
    )(xt)


def kernel(inputs):
    xt = jnp.swapaxes(inputs, 0, 1)
    return _argmax_tc(xt).reshape(R)


# speedup vs baseline: 7.6368x; 3.3998x over previous
import jax
import jax.numpy as jnp
from jax import lax
from jax.experimental import pallas as pl
from jax.experimental.pallas import tpu as pltpu

R, C = 1024, 100000
SUB = 8
RT = 800
NJ = C // RT
_NEG_INF = float("-inf")
_BIG = 1 << 30


def _tc_body(x_ref, o_ref, m_ref, c_ref):
    j = pl.program_id(0)

    @pl.when(j == 0)
    def _():
        m_ref[...] = jnp.full((SUB, R), _NEG_INF, dtype=jnp.float32)
        c_ref[...] = jnp.zeros((SUB, R), dtype=jnp.int32)

    m = m_ref[...]
    c = c_ref[...]
    for k in range(RT // SUB):
        v = x_ref[pl.ds(SUB * k, SUB), :]
        p = v > m
        m = jnp.where(p, v, m)
        c = jnp.where(p, j * RT + SUB * k, c)
    m_ref[...] = m
    c_ref[...] = c

    @pl.when(j == NJ - 1)
    def _():
        mm = jnp.max(m, axis=0, keepdims=True)
        srow = lax.broadcasted_iota(jnp.int32, (SUB, R), 0)
        cand = jnp.where(m == mm, c + srow, _BIG)
        o_ref[...] = jnp.min(cand, axis=0, keepdims=True)


def _argmax_tc(xt):
    return pl.pallas_call(
        _tc_body,
        grid=(NJ,),
        in_specs=[pl.BlockSpec((RT, R), lambda j: (j, 0))],
        out_specs=pl.BlockSpec((1, R), lambda j: (0, 0)),
        out_shape=jax.ShapeDtypeStruct((1, R), jnp.int32),
        scratch_shapes=[
            pltpu.VMEM((SUB, R), jnp.float32),
            pltpu.VMEM((SUB, R), jnp.int32),
        ],
        compiler_params=pltpu.CompilerParams(
            dimension_semantics=("arbitrary",),
        ),
    )(xt)


def kernel(inputs):
    xt = jnp.swapaxes(inputs, 0, 1)
    return _argmax_tc(xt).reshape(R)


# speedup vs baseline: 8.4353x; 1.1046x over previous
import functools

import jax
import jax.numpy as jnp
from jax import lax
from jax.experimental import pallas as pl
from jax.experimental.pallas import tpu as pltpu
from jax.experimental.pallas import tpu_sc as plsc

R, C = 1024, 100000
SUB = 8
LANE = 128
_NEG_INF = float("-inf")
_BIG = 1 << 30

C_SC = 32000
C_TC = C - C_SC

RT = 800
NJ = C_TC // RT

NCORES, NSUB = 2, 16
NW = NCORES * NSUB
NR4 = 4
NLB = 8
RPW = C_SC // NR4
RTS = 200
NCH = RPW // RTS
PAIRS = NCH // 2


def _tc_body(x_ref, ov_ref, oi_ref, m_ref, c_ref):
    j = pl.program_id(0)

    @pl.when(j == 0)
    def _():
        m_ref[...] = jnp.full((SUB, R), _NEG_INF, dtype=jnp.float32)
        c_ref[...] = jnp.zeros((SUB, R), dtype=jnp.int32)

    m = m_ref[...]
    c = c_ref[...]
    for k in range(RT // SUB):
        v = x_ref[pl.ds(SUB * k, SUB), :]
        p = v > m
        m = jnp.where(p, v, m)
        c = jnp.where(p, j * RT + SUB * k, c)
    m_ref[...] = m
    c_ref[...] = c

    @pl.when(j == NJ - 1)
    def _():
        mm = jnp.max(m, axis=0, keepdims=True)
        srow = lax.broadcasted_iota(jnp.int32, (SUB, R), 0)
        cand = jnp.where(m == mm, c + srow, _BIG)
        ov_ref[...] = mm
        oi_ref[...] = jnp.min(cand, axis=0, keepdims=True)


def _argmax_tc(xt):
    return pl.pallas_call(
        _tc_body,
        grid=(NJ,),
        in_specs=[pl.BlockSpec((RT, R), lambda j: (j, 0))],
        out_specs=[
            pl.BlockSpec((1, R), lambda j: (0, 0)),
            pl.BlockSpec((1, R), lambda j: (0, 0)),
        ],
        out_shape=[
            jax.ShapeDtypeStruct((1, R), jnp.float32),
            jax.ShapeDtypeStruct((1, R), jnp.int32),
        ],
        scratch_shapes=[
            pltpu.VMEM((SUB, R), jnp.float32),
            pltpu.VMEM((SUB, R), jnp.int32),
        ],
        compiler_params=pltpu.CompilerParams(
            dimension_semantics=("arbitrary",),
        ),
    )(xt)


def _partial_sc(xt):
    mesh = plsc.VectorSubcoreMesh(core_axis_name="c", subcore_axis_name="s")

    @functools.partial(
        pl.kernel,
        out_type=(
            jax.ShapeDtypeStruct((NR4, NLB, LANE), jnp.float32),
            jax.ShapeDtypeStruct((NR4, NLB, LANE), jnp.int32),
        ),
        mesh=mesh,
        compiler_params=pltpu.CompilerParams(
            needs_layout_passes=False, use_tc_tiling_on_sc=True
        ),
        scratch_types=[
            pltpu.VMEM((2, RTS, LANE), jnp.float32),
            pltpu.VMEM((LANE,), jnp.float32),
            pltpu.VMEM((LANE,), jnp.int32),
            pltpu.SemaphoreType.DMA,
            pltpu.SemaphoreType.DMA,
        ],
    )
    def k(xt_hbm, val_hbm, idx_hbm, buf, vstage, istage, sem0, sem1):
        cid = lax.axis_index("c")
        sid = lax.axis_index("s")
        wid = sid * NCORES + cid
        lb = wid % NLB
        r4 = wid // NLB
        row0 = C_TC + r4 * RPW

        def chunk_copy(ci, slot):
            sem = sem0 if slot == 0 else sem1
            src = xt_hbm.at[pl.ds(row0 + ci * RTS, RTS), pl.ds(lb * LANE, LANE)]
            return pltpu.make_async_copy(src, buf.at[slot], sem)

        chunk_copy(0, 0).start()
        chunk_copy(1, 1).start()

        neg = jnp.full((16,), _NEG_INF, dtype=jnp.float32)
        zero = jnp.zeros((16,), dtype=jnp.int32)

        def compute_chunk(slot, colbase, st):
            def body(rt, st_):
                out = list(st_)
                for s in range(SUB):
                    col = colbase + rt * SUB + s
                    for kk in range(8):
                        rm, rc = out[2 * kk], out[2 * kk + 1]
                        v = buf[slot, rt * SUB + s, pl.ds(kk * 16, 16)]
                        p = v > rm
                        out[2 * kk] = jnp.where(p, v, rm)
                        out[2 * kk + 1] = jnp.where(p, col, rc)
                return tuple(out)

            return lax.fori_loop(0, RTS // SUB, body, tuple(st))

        def pair_body(t, st):
            c0 = 2 * t
            chunk_copy(c0, 0).wait()
            st = compute_chunk(0, row0 + c0 * RTS, st)

            @pl.when(t < PAIRS - 1)
            def _():
                chunk_copy(c0 + 2, 0).start()

            chunk_copy(c0 + 1, 1).wait()
            st = compute_chunk(1, row0 + (c0 + 1) * RTS, st)

            @pl.when(t < PAIRS - 1)
            def _():
                chunk_copy(c0 + 3, 1).start()

            return st

        st = lax.fori_loop(0, PAIRS, pair_body, (neg, zero) * 8)
        for kk in range(8):
            vstage[pl.ds(kk * 16, 16)] = st[2 * kk]
            istage[pl.ds(kk * 16, 16)] = st[2 * kk + 1]
        pltpu.sync_copy(vstage, val_hbm.at[r4, lb])
        pltpu.sync_copy(istage, idx_hbm.at[r4, lb])

    return k(xt)


def _merge_body(tv_ref, ti_ref, sv_ref, si_ref, o_ref):
    bv = tv_ref[...]
    bi = ti_ref[...]
    for r in range(NR4):
        sv = sv_ref[pl.ds(r, 1), :]
        si = si_ref[pl.ds(r, 1), :]
        p = sv > bv
        bv = jnp.where(p, sv, bv)
        bi = jnp.where(p, si, bi)
    o_ref[...] = bi


def _merge(tc_val, tc_idx, sc_val, sc_idx):
    return pl.pallas_call(
        _merge_body,
        out_shape=jax.ShapeDtypeStruct((1, R), jnp.int32),
    )(tc_val, tc_idx, sc_val, sc_idx)


def kernel(inputs):
    xt = jnp.swapaxes(inputs, 0, 1)
    sc_val, sc_idx = _partial_sc(xt)
    tc_val, tc_idx = _argmax_tc(xt)
    out = _merge(
        tc_val, tc_idx, sc_val.reshape(NR4, R), sc_idx.reshape(NR4, R)
    )
    return out.reshape(R)


# speedup vs baseline: 8.8482x; 1.0489x over previous
import functools

import jax
import jax.numpy as jnp
from jax import lax
from jax.experimental import pallas as pl
from jax.experimental.pallas import tpu as pltpu
from jax.experimental.pallas import tpu_sc as plsc

R, C = 1024, 100000
SUB = 8
LANE = 128
_NEG_INF = float("-inf")
_BIG = 1 << 30

C_SC = 40000
C_TC = C - C_SC

RT = 800
NJ = C_TC // RT

NCORES, NSUB = 2, 16
NW = NCORES * NSUB
NR4 = 4
NLB = 8
RPW = C_SC // NR4
RTS = 200
NCH = RPW // RTS
PAIRS = NCH // 2


def _tc_body(x_ref, ov_ref, oi_ref, m_ref, c_ref):
    j = pl.program_id(0)

    @pl.when(j == 0)
    def _():
        m_ref[...] = jnp.full((SUB, R), _NEG_INF, dtype=jnp.float32)
        c_ref[...] = jnp.zeros((SUB, R), dtype=jnp.int32)

    m = m_ref[...]
    c = c_ref[...]
    for k in range(RT // SUB):
        v = x_ref[pl.ds(SUB * k, SUB), :]
        p = v > m
        m = jnp.where(p, v, m)
        c = jnp.where(p, j * RT + SUB * k, c)
    m_ref[...] = m
    c_ref[...] = c

    @pl.when(j == NJ - 1)
    def _():
        mm = jnp.max(m, axis=0, keepdims=True)
        srow = lax.broadcasted_iota(jnp.int32, (SUB, R), 0)
        cand = jnp.where(m == mm, c + srow, _BIG)
        ov_ref[...] = mm
        oi_ref[...] = jnp.min(cand, axis=0, keepdims=True)


def _argmax_tc(xt):
    return pl.pallas_call(
        _tc_body,
        grid=(NJ,),
        in_specs=[pl.BlockSpec((RT, R), lambda j: (j, 0))],
        out_specs=[
            pl.BlockSpec((1, R), lambda j: (0, 0)),
            pl.BlockSpec((1, R), lambda j: (0, 0)),
        ],
        out_shape=[
            jax.ShapeDtypeStruct((1, R), jnp.float32),
            jax.ShapeDtypeStruct((1, R), jnp.int32),
        ],
        scratch_shapes=[
            pltpu.VMEM((SUB, R), jnp.float32),
            pltpu.VMEM((SUB, R), jnp.int32),
        ],
        compiler_params=pltpu.CompilerParams(
            dimension_semantics=("arbitrary",),
        ),
    )(xt)


def _partial_sc(xt):
    mesh = plsc.VectorSubcoreMesh(core_axis_name="c", subcore_axis_name="s")

    @functools.partial(
        pl.kernel,
        out_type=(
            jax.ShapeDtypeStruct((NR4, NLB, LANE), jnp.float32),
            jax.ShapeDtypeStruct((NR4, NLB, LANE), jnp.int32),
        ),
        mesh=mesh,
        compiler_params=pltpu.CompilerParams(
            needs_layout_passes=False, use_tc_tiling_on_sc=True
        ),
        scratch_types=[
            pltpu.VMEM((2, RTS, LANE), jnp.float32),
            pltpu.VMEM((LANE,), jnp.float32),
            pltpu.VMEM((LANE,), jnp.int32),
            pltpu.SemaphoreType.DMA,
            pltpu.SemaphoreType.DMA,
        ],
    )
    def k(xt_hbm, val_hbm, idx_hbm, buf, vstage, istage, sem0, sem1):
        cid = lax.axis_index("c")
        sid = lax.axis_index("s")
        wid = sid * NCORES + cid
        lb = wid % NLB
        r4 = wid // NLB
        row0 = C_TC + r4 * RPW

        def chunk_copy(ci, slot):
            sem = sem0 if slot == 0 else sem1
            src = xt_hbm.at[pl.ds(row0 + ci * RTS, RTS), pl.ds(lb * LANE, LANE)]
            return pltpu.make_async_copy(src, buf.at[slot], sem)

        chunk_copy(0, 0).start()
        chunk_copy(1, 1).start()

        neg = jnp.full((16,), _NEG_INF, dtype=jnp.float32)
        zero = jnp.zeros((16,), dtype=jnp.int32)

        def compute_chunk(slot, colbase, st):
            def body(rt, st_):
                out = list(st_)
                for s in range(SUB):
                    col = colbase + rt * SUB + s
                    for kk in range(8):
                        rm, rc = out[2 * kk], out[2 * kk + 1]
                        v = buf[slot, rt * SUB + s, pl.ds(kk * 16, 16)]
                        p = v > rm
                        out[2 * kk] = jnp.where(p, v, rm)
                        out[2 * kk + 1] = jnp.where(p, col, rc)
                return tuple(out)

            return lax.fori_loop(0, RTS // SUB, body, tuple(st))

        def pair_body(t, st):
            c0 = 2 * t
            chunk_copy(c0, 0).wait()
            st = compute_chunk(0, row0 + c0 * RTS, st)

            @pl.when(t < PAIRS - 1)
            def _():
                chunk_copy(c0 + 2, 0).start()

            chunk_copy(c0 + 1, 1).wait()
            st = compute_chunk(1, row0 + (c0 + 1) * RTS, st)

            @pl.when(t < PAIRS - 1)
            def _():
                chunk_copy(c0 + 3, 1).start()

            return st

        st = lax.fori_loop(0, PAIRS, pair_body, (neg, zero) * 8)
        for kk in range(8):
            vstage[pl.ds(kk * 16, 16)] = st[2 * kk]
            istage[pl.ds(kk * 16, 16)] = st[2 * kk + 1]
        pltpu.sync_copy(vstage, val_hbm.at[r4, lb])
        pltpu.sync_copy(istage, idx_hbm.at[r4, lb])

    return k(xt)


def _merge_body(tv_ref, ti_ref, sv_ref, si_ref, o_ref):
    bv = tv_ref[...]
    bi = ti_ref[...]
    for r in range(NR4):
        sv = sv_ref[pl.ds(r, 1), :]
        si = si_ref[pl.ds(r, 1), :]
        p = sv > bv
        bv = jnp.where(p, sv, bv)
        bi = jnp.where(p, si, bi)
    o_ref[...] = bi


def _merge(tc_val, tc_idx, sc_val, sc_idx):
    return pl.pallas_call(
        _merge_body,
        out_shape=jax.ShapeDtypeStruct((1, R), jnp.int32),
    )(tc_val, tc_idx, sc_val, sc_idx)


def kernel(inputs):
    xt = jnp.swapaxes(inputs, 0, 1)
    sc_val, sc_idx = _partial_sc(xt)
    tc_val, tc_idx = _argmax_tc(xt)
    out = _merge(
        tc_val, tc_idx, sc_val.reshape(NR4, R), sc_idx.reshape(NR4, R)
    )
    return out.reshape(R)


# speedup vs baseline: 9.0772x; 1.0259x over previous
import functools

import jax
import jax.numpy as jnp
from jax import lax
from jax.experimental import pallas as pl
from jax.experimental.pallas import tpu as pltpu
from jax.experimental.pallas import tpu_sc as plsc

R, C = 1024, 100000
SUB = 8
LANE = 128
_NEG_INF = float("-inf")
_BIG = 1 << 30

C_SC = 40000
C_TC = C - C_SC

RT = 2000
NJ = C_TC // RT

NCORES, NSUB = 2, 16
NW = NCORES * NSUB
NR4 = 4
NLB = 8
RPW = C_SC // NR4
RTS = 200
NCH = RPW // RTS
PAIRS = NCH // 2


def _tc_body(x_ref, ov_ref, oi_ref, m_ref, c_ref):
    j = pl.program_id(0)

    @pl.when(j == 0)
    def _():
        m_ref[...] = jnp.full((SUB, R), _NEG_INF, dtype=jnp.float32)
        c_ref[...] = jnp.zeros((SUB, R), dtype=jnp.int32)

    m = m_ref[...]
    c = c_ref[...]
    for k in range(RT // SUB):
        v = x_ref[pl.ds(SUB * k, SUB), :]
        p = v > m
        m = jnp.where(p, v, m)
        c = jnp.where(p, j * RT + SUB * k, c)
    m_ref[...] = m
    c_ref[...] = c

    @pl.when(j == NJ - 1)
    def _():
        mm = jnp.max(m, axis=0, keepdims=True)
        srow = lax.broadcasted_iota(jnp.int32, (SUB, R), 0)
        cand = jnp.where(m == mm, c + srow, _BIG)
        ov_ref[...] = mm
        oi_ref[...] = jnp.min(cand, axis=0, keepdims=True)


def _argmax_tc(xt):
    return pl.pallas_call(
        _tc_body,
        grid=(NJ,),
        in_specs=[pl.BlockSpec((RT, R), lambda j: (j, 0))],
        out_specs=[
            pl.BlockSpec((1, R), lambda j: (0, 0)),
            pl.BlockSpec((1, R), lambda j: (0, 0)),
        ],
        out_shape=[
            jax.ShapeDtypeStruct((1, R), jnp.float32),
            jax.ShapeDtypeStruct((1, R), jnp.int32),
        ],
        scratch_shapes=[
            pltpu.VMEM((SUB, R), jnp.float32),
            pltpu.VMEM((SUB, R), jnp.int32),
        ],
        compiler_params=pltpu.CompilerParams(
            dimension_semantics=("arbitrary",),
        ),
    )(xt)


def _partial_sc(xt):
    mesh = plsc.VectorSubcoreMesh(core_axis_name="c", subcore_axis_name="s")

    @functools.partial(
        pl.kernel,
        out_type=(
            jax.ShapeDtypeStruct((NR4, NLB, LANE), jnp.float32),
            jax.ShapeDtypeStruct((NR4, NLB, LANE), jnp.int32),
        ),
        mesh=mesh,
        compiler_params=pltpu.CompilerParams(
            needs_layout_passes=False, use_tc_tiling_on_sc=True
        ),
        scratch_types=[
            pltpu.VMEM((2, RTS, LANE), jnp.float32),
            pltpu.VMEM((LANE,), jnp.float32),
            pltpu.VMEM((LANE,), jnp.int32),
            pltpu.SemaphoreType.DMA,
            pltpu.SemaphoreType.DMA,
        ],
    )
    def k(xt_hbm, val_hbm, idx_hbm, buf, vstage, istage, sem0, sem1):
        cid = lax.axis_index("c")
        sid = lax.axis_index("s")
        wid = sid * NCORES + cid
        lb = wid % NLB
        r4 = wid // NLB
        row0 = C_TC + r4 * RPW

        def chunk_copy(ci, slot):
            sem = sem0 if slot == 0 else sem1
            src = xt_hbm.at[pl.ds(row0 + ci * RTS, RTS), pl.ds(lb * LANE, LANE)]
            return pltpu.make_async_copy(src, buf.at[slot], sem)

        chunk_copy(0, 0).start()
        chunk_copy(1, 1).start()

        neg = jnp.full((16,), _NEG_INF, dtype=jnp.float32)
        zero = jnp.zeros((16,), dtype=jnp.int32)

        def compute_chunk(slot, colbase, st):
            def body(rt, st_):
                out = list(st_)
                for s in range(SUB):
                    col = colbase + rt * SUB + s
                    for kk in range(8):
                        rm, rc = out[2 * kk], out[2 * kk + 1]
                        v = buf[slot, rt * SUB + s, pl.ds(kk * 16, 16)]
                        p = v > rm
                        out[2 * kk] = jnp.where(p, v, rm)
                        out[2 * kk + 1] = jnp.where(p, col, rc)
                return tuple(out)

            return lax.fori_loop(0, RTS // SUB, body, tuple(st))

        def pair_body(t, st):
            c0 = 2 * t
            chunk_copy(c0, 0).wait()
            st = compute_chunk(0, row0 + c0 * RTS, st)

            @pl.when(t < PAIRS - 1)
            def _():
                chunk_copy(c0 + 2, 0).start()

            chunk_copy(c0 + 1, 1).wait()
            st = compute_chunk(1, row0 + (c0 + 1) * RTS, st)

            @pl.when(t < PAIRS - 1)
            def _():
                chunk_copy(c0 + 3, 1).start()

            return st

        st = lax.fori_loop(0, PAIRS, pair_body, (neg, zero) * 8)
        for kk in range(8):
            vstage[pl.ds(kk * 16, 16)] = st[2 * kk]
            istage[pl.ds(kk * 16, 16)] = st[2 * kk + 1]
        pltpu.sync_copy(vstage, val_hbm.at[r4, lb])
        pltpu.sync_copy(istage, idx_hbm.at[r4, lb])

    return k(xt)


def _merge_body(tv_ref, ti_ref, sv_ref, si_ref, o_ref):
    bv = tv_ref[...]
    bi = ti_ref[...]
    for r in range(NR4):
        sv = sv_ref[pl.ds(r, 1), :]
        si = si_ref[pl.ds(r, 1), :]
        p = sv > bv
        bv = jnp.where(p, sv, bv)
        bi = jnp.where(p, si, bi)
    o_ref[...] = bi


def _merge(tc_val, tc_idx, sc_val, sc_idx):
    return pl.pallas_call(
        _merge_body,
        out_shape=jax.ShapeDtypeStruct((1, R), jnp.int32),
    )(tc_val, tc_idx, sc_val, sc_idx)


def kernel(inputs):
    xt = jnp.swapaxes(inputs, 0, 1)
    sc_val, sc_idx = _partial_sc(xt)
    tc_val, tc_idx = _argmax_tc(xt)
    out = _merge(
        tc_val, tc_idx, sc_val.reshape(NR4, R), sc_idx.reshape(NR4, R)
    )
    return out.reshape(R)
